# Initial kernel scaffold; baseline (speedup 1.0000x reference)
import functools

import jax
import jax.numpy as jnp
from jax import lax
from jax.experimental import pallas as pl
from jax.experimental.pallas import tpu as pltpu
from jax.experimental.pallas import tpu_sc as plsc

N = 10000
D = 128
E = 320000
NC = 2
NS = 16
NW = NC * NS
L = 16
CH = 128
CHUNKS = 80
EPW = CH * CHUNKS
E_PAD = NW * EPW
RPS = N // NS
BM = 1000


def _mm_body(x_ref, w_ref, o_ref):
    o_ref[...] = jnp.dot(x_ref[...], w_ref[...],
                         preferred_element_type=jnp.float32)


def _matmul(x, W):
    return pl.pallas_call(
        _mm_body,
        grid=(N // BM,),
        in_specs=[pl.BlockSpec((BM, D), lambda i: (i, 0)),
                  pl.BlockSpec((D, D), lambda i: (0, 0))],
        out_specs=pl.BlockSpec((BM, D), lambda i: (i, 0)),
        out_shape=jax.ShapeDtypeStruct((N, D), jnp.float32),
    )(x, W)


def _fused_mm_body(p_ref, b_ref, w_ref, o_ref):
    h = jnp.maximum(p_ref[0] + p_ref[1] + b_ref[...], 0.0)
    o_ref[...] = jnp.dot(h, w_ref[...], preferred_element_type=jnp.float32)


def _relu_bias_matmul(p, b, W):
    return pl.pallas_call(
        _fused_mm_body,
        grid=(N // BM,),
        in_specs=[pl.BlockSpec((NC, BM, D), lambda i: (0, i, 0)),
                  pl.BlockSpec((1, D), lambda i: (0, 0)),
                  pl.BlockSpec((D, D), lambda i: (0, 0))],
        out_specs=pl.BlockSpec((BM, D), lambda i: (i, 0)),
        out_shape=jax.ShapeDtypeStruct((N, D), jnp.float32),
    )(p, b, W)


def _bias_add_body(p_ref, b_ref, o_ref):
    o_ref[...] = p_ref[0] + p_ref[1] + b_ref[...]


def _bias_add(p, b):
    return pl.pallas_call(
        _bias_add_body,
        grid=(N // BM,),
        in_specs=[pl.BlockSpec((NC, BM, D), lambda i: (0, i, 0)),
                  pl.BlockSpec((1, D), lambda i: (0, 0))],
        out_specs=pl.BlockSpec((BM, D), lambda i: (i, 0)),
        out_shape=jax.ShapeDtypeStruct((N, D), jnp.float32),
    )(p, b)


_MESH = plsc.VectorSubcoreMesh(core_axis_name="c", subcore_axis_name="s")


@functools.partial(
    pl.kernel,
    out_type=jax.ShapeDtypeStruct((NC, N, D), jnp.float32),
    mesh=_MESH,
    scratch_types=[
        pltpu.VMEM((EPW,), jnp.int32),
        pltpu.VMEM((CHUNKS, CH), jnp.int32),
        pltpu.VMEM((CH, L), jnp.float32),
        pltpu.VMEM((CH, D), jnp.float32),
        pltpu.VMEM_SHARED((N, D), jnp.float32),
    ],
)
def _spmm_kernel(dense_hbm, col_hbm, row_hbm, wexp_hbm, out_hbm,
                 col_v, row_v, w_v, rows_v, acc):
    cid = lax.axis_index("c")
    sid = lax.axis_index("s")
    wid = sid * NC + cid

    @pl.loop(0, CH)
    def _(r):
        for t in range(D // L):
            rows_v[r, pl.ds(t * L, L)] = jnp.zeros((L,), jnp.float32)

    @pl.loop(0, 5)
    def _(i):
        pltpu.sync_copy(rows_v.at[pl.ds(0, 125)],
                        acc.at[pl.ds(sid * RPS + i * 125, 125)])

    plsc.subcore_barrier()

    pltpu.sync_copy(col_hbm.at[wid], col_v)
    pltpu.sync_copy(row_hbm.at[wid], row_v)

    @pl.loop(0, CHUNKS)
    def _(c):
        pltpu.sync_copy(wexp_hbm.at[wid].at[c], w_v)
        pltpu.sync_copy(dense_hbm.at[col_v.at[pl.ds(c * CH, CH)]], rows_v)

        @pl.loop(0, CH)
        def _(e):
            wvec = w_v[e, :]
            for t in range(D // L):
                rows_v[e, pl.ds(t * L, L)] = rows_v[e, pl.ds(t * L, L)] * wvec

        pltpu.sync_copy(rows_v, acc.at[row_v.at[c]], add=True)

    plsc.subcore_barrier()

    @pl.loop(0, 5)
    def _(i):
        pltpu.sync_copy(acc.at[pl.ds(sid * RPS + i * 125, 125)],
                        out_hbm.at[cid].at[pl.ds(sid * RPS + i * 125, 125)])


def _spmm_partials(dense, colp, rowp, wexp):
    return _spmm_kernel(dense, colp, rowp, wexp)


def kernel(x, edge_index, edge_weight, W1, b1, W2, b2):
    row = edge_index[0].astype(jnp.int32)
    col = edge_index[1].astype(jnp.int32)
    w = edge_weight.astype(jnp.float32)

    pad = E_PAD - E
    zi = jnp.zeros((pad,), jnp.int32)
    colp = jnp.concatenate([col, zi]).reshape(NW, EPW)
    rowp = jnp.concatenate([row, zi]).reshape(NW, CHUNKS, CH)
    wp = jnp.concatenate([w, jnp.zeros((pad,), jnp.float32)])
    wexp = jnp.broadcast_to(wp[:, None], (E_PAD, L)).reshape(NW, CHUNKS, CH, L)

    support = _matmul(x, W1)
    p1 = _spmm_partials(support, colp, rowp, wexp)
    support2 = _relu_bias_matmul(p1, b1.reshape(1, D), W2)
    p2 = _spmm_partials(support2, colp, rowp, wexp)
    return _bias_add(p2, b2.reshape(1, D))


# speedup vs baseline: 2.2964x; 2.2964x over previous
import functools

import jax
import jax.numpy as jnp
from jax import lax
from jax.experimental import pallas as pl
from jax.experimental.pallas import tpu as pltpu
from jax.experimental.pallas import tpu_sc as plsc

N = 10000
D = 128
E = 320000
NC = 2
NS = 16
NW = NC * NS
L = 16
CH = 128
CHUNKS = 80
EPW = CH * CHUNKS
E_PAD = NW * EPW
N_PAD = 10240
RPS = N_PAD // NS
BM = 1000


def _mm_body(x_ref, w_ref, o_ref):
    o_ref[...] = jnp.dot(x_ref[...], w_ref[...],
                         preferred_element_type=jnp.float32)


def _matmul(x, W):
    return pl.pallas_call(
        _mm_body,
        grid=(N // BM,),
        in_specs=[pl.BlockSpec((BM, D), lambda i: (i, 0)),
                  pl.BlockSpec((D, D), lambda i: (0, 0))],
        out_specs=pl.BlockSpec((BM, D), lambda i: (i, 0)),
        out_shape=jax.ShapeDtypeStruct((N, D), jnp.float32),
    )(x, W)


def _fused_mm_body(p_ref, b_ref, w_ref, o_ref):
    h = jnp.maximum(p_ref[0] + p_ref[1] + b_ref[0][None, :], 0.0)
    o_ref[...] = jnp.dot(h, w_ref[...], preferred_element_type=jnp.float32)


def _relu_bias_matmul(p, b, W):
    return pl.pallas_call(
        _fused_mm_body,
        grid=(N // BM,),
        in_specs=[pl.BlockSpec((NC, BM, D), lambda i: (0, i, 0)),
                  pl.BlockSpec((8, D), lambda i: (0, 0)),
                  pl.BlockSpec((D, D), lambda i: (0, 0))],
        out_specs=pl.BlockSpec((BM, D), lambda i: (i, 0)),
        out_shape=jax.ShapeDtypeStruct((N, D), jnp.float32),
    )(p, b, W)


def _bias_add_body(p_ref, b_ref, o_ref):
    o_ref[...] = p_ref[0] + p_ref[1] + b_ref[0][None, :]


def _bias_add(p, b):
    return pl.pallas_call(
        _bias_add_body,
        grid=(N // BM,),
        in_specs=[pl.BlockSpec((NC, BM, D), lambda i: (0, i, 0)),
                  pl.BlockSpec((8, D), lambda i: (0, 0))],
        out_specs=pl.BlockSpec((BM, D), lambda i: (i, 0)),
        out_shape=jax.ShapeDtypeStruct((N, D), jnp.float32),
    )(p, b)


_MESH = plsc.VectorSubcoreMesh(core_axis_name="c", subcore_axis_name="s")


@functools.partial(
    pl.kernel,
    out_type=jax.ShapeDtypeStruct((NC, N_PAD, D), jnp.float32),
    mesh=_MESH,
    scratch_types=[
        pltpu.VMEM((CH,), jnp.int32),
        pltpu.VMEM((CH,), jnp.int32),
        pltpu.VMEM((CH, L), jnp.float32),
        pltpu.VMEM((CH, D), jnp.float32),
        pltpu.VMEM_SHARED((N_PAD, D), jnp.float32),
    ],
)
def _spmm_kernel(dense_hbm, col_hbm, row_hbm, wexp_hbm, out_hbm,
                 col_v, row_v, w_v, rows_v, acc):
    cid = lax.axis_index("c")
    sid = lax.axis_index("s")
    wid = sid * NC + cid

    @pl.loop(0, CH)
    def _(r):
        for t in range(D // L):
            rows_v[r, pl.ds(t * L, L)] = jnp.zeros((L,), jnp.float32)

    @pl.loop(0, RPS // CH)
    def _(i):
        pltpu.sync_copy(rows_v,
                        acc.at[pl.ds(sid * RPS + i * CH, CH)])

    plsc.subcore_barrier()

    @pl.loop(0, CHUNKS)
    def _(c):
        pltpu.sync_copy(col_hbm.at[wid].at[c], col_v)
        pltpu.sync_copy(row_hbm.at[wid].at[c], row_v)
        pltpu.sync_copy(wexp_hbm.at[wid].at[c], w_v)
        pltpu.sync_copy(dense_hbm.at[col_v], rows_v)

        @pl.loop(0, CH)
        def _(e):
            wvec = w_v[e, :]
            for t in range(D // L):
                rows_v[e, pl.ds(t * L, L)] = rows_v[e, pl.ds(t * L, L)] * wvec

        pltpu.sync_copy(rows_v, acc.at[row_v], add=True)

    plsc.subcore_barrier()

    @pl.loop(0, RPS // CH)
    def _(i):
        pltpu.sync_copy(acc.at[pl.ds(sid * RPS + i * CH, CH)],
                        out_hbm.at[cid].at[pl.ds(sid * RPS + i * CH, CH)])


def _spmm_partials(dense, colp, rowp, wexp):
    return _spmm_kernel(dense, colp, rowp, wexp)


def kernel(x, edge_index, edge_weight, W1, b1, W2, b2):
    row = edge_index[0].astype(jnp.int32)
    col = edge_index[1].astype(jnp.int32)
    w = edge_weight.astype(jnp.float32)

    pad = E_PAD - E
    zi = jnp.zeros((pad,), jnp.int32)
    colp = jnp.concatenate([col, zi]).reshape(NW, CHUNKS, CH)
    rowp = jnp.concatenate([row, zi]).reshape(NW, CHUNKS, CH)
    wp = jnp.concatenate([w, jnp.zeros((pad,), jnp.float32)])
    wexp = jnp.broadcast_to(wp[:, None], (E_PAD, L)).reshape(NW, CHUNKS, CH, L)

    b1e = jnp.broadcast_to(b1[None, :], (8, D))
    b2e = jnp.broadcast_to(b2[None, :], (8, D))

    support = _matmul(x, W1)
    p1 = _spmm_partials(support, colp, rowp, wexp)
    support2 = _relu_bias_matmul(p1, b1e, W2)
    p2 = _spmm_partials(support2, colp, rowp, wexp)
    return _bias_add(p2, b2e)


# speedup vs baseline: 3.6723x; 1.5992x over previous
import dataclasses
import functools

import jax
import jax.numpy as jnp
from jax import lax
from jax.experimental import pallas as pl
from jax.experimental.pallas import tpu as pltpu
from jax.experimental.pallas import tpu_sc as plsc

N = 10000
D = 128
E = 320000
NC = 2
NS = 16
NW = NC * NS
L = 16
CH = 128
CHUNKS = 80
EPW = CH * CHUNKS
E_PAD = NW * EPW
RPS = 624
BM = 1000

_GDN = lax.GatherDimensionNumbers(
    offset_dims=(), collapsed_slice_dims=(0,), start_index_map=(0,))


def _mm_body(x_ref, w_ref, o_ref):
    o_ref[...] = jnp.dot(x_ref[...], w_ref[...],
                         preferred_element_type=jnp.float32)


def _matmul(x, W):
    return pl.pallas_call(
        _mm_body,
        grid=(N // BM,),
        in_specs=[pl.BlockSpec((BM, D), lambda i: (i, 0)),
                  pl.BlockSpec((D, D), lambda i: (0, 0))],
        out_specs=pl.BlockSpec((BM, D), lambda i: (i, 0)),
        out_shape=jax.ShapeDtypeStruct((N, D), jnp.float32),
    )(x, W)


def _fused_mm_body(p_ref, b_ref, w_ref, o_ref):
    h = jnp.maximum(p_ref[0] + p_ref[1] + b_ref[0][None, :], 0.0)
    o_ref[...] = jnp.dot(h, w_ref[...], preferred_element_type=jnp.float32)


def _relu_bias_matmul(p, b, W):
    return pl.pallas_call(
        _fused_mm_body,
        grid=(N // BM,),
        in_specs=[pl.BlockSpec((NC, BM, D), lambda i: (0, i, 0)),
                  pl.BlockSpec((8, D), lambda i: (0, 0)),
                  pl.BlockSpec((D, D), lambda i: (0, 0))],
        out_specs=pl.BlockSpec((BM, D), lambda i: (i, 0)),
        out_shape=jax.ShapeDtypeStruct((N, D), jnp.float32),
    )(p, b, W)


def _bias_add_body(p_ref, b_ref, o_ref):
    o_ref[...] = p_ref[0] + p_ref[1] + b_ref[0][None, :]


def _bias_add(p, b):
    return pl.pallas_call(
        _bias_add_body,
        grid=(N // BM,),
        in_specs=[pl.BlockSpec((NC, BM, D), lambda i: (0, i, 0)),
                  pl.BlockSpec((8, D), lambda i: (0, 0))],
        out_specs=pl.BlockSpec((BM, D), lambda i: (i, 0)),
        out_shape=jax.ShapeDtypeStruct((N, D), jnp.float32),
    )(p, b)


_MESH = plsc.VectorSubcoreMesh(core_axis_name="c", subcore_axis_name="s")

_CP = pltpu.CompilerParams()
if "needs_layout_passes" in pltpu.CompilerParams.__dataclass_fields__:
    _CP = dataclasses.replace(_CP, needs_layout_passes=False)


@functools.partial(
    pl.kernel,
    out_type=jax.ShapeDtypeStruct((NC, N, D), jnp.float32),
    mesh=_MESH,
    compiler_params=_CP,
    scratch_types=[
        pltpu.VMEM((2, CH), jnp.int32),
        pltpu.VMEM((2, CH), jnp.int32),
        pltpu.VMEM((1, CH), jnp.int32),
        pltpu.VMEM((1, CH), jnp.int32),
        pltpu.VMEM((CH, D), jnp.float32),
        pltpu.VMEM((CH, D), jnp.float32),
        pltpu.SemaphoreType.DMA,
        pltpu.SemaphoreType.DMA,
        pltpu.SemaphoreType.DMA,
        pltpu.SemaphoreType.DMA,
        pltpu.SemaphoreType.DMA,
        pltpu.SemaphoreType.DMA,
        pltpu.SemaphoreType.DMA,
        pltpu.SemaphoreType.DMA,
        pltpu.VMEM_SHARED((N, D), jnp.float32),
    ],
)
def _spmm_kernel(dense_hbm, cw_hbm, row_hbm, out_hbm,
                 cw0, cw1, ri0, ri1, r0, r1,
                 csem0, csem1, rsem0, rsem1, gsem0, gsem1, ssem0, ssem1,
                 acc):
    cid = lax.axis_index("c")
    sid = lax.axis_index("s")
    wid = sid * NC + cid

    cwbuf = (cw0, cw1)
    ribuf = (ri0, ri1)
    rbuf = (r0, r1)
    csem = (csem0, csem1)
    rsem = (rsem0, rsem1)
    gsem = (gsem0, gsem1)
    ssem = (ssem0, ssem1)

    @pl.loop(0, CH)
    def _(r):
        for t in range(D // L):
            r0[r, pl.ds(t * L, L)] = jnp.zeros((L,), jnp.float32)

    base = sid * RPS
    for k in range(4):
        pltpu.sync_copy(r0, acc.at[pl.ds(base + k * CH, CH)])
    pltpu.sync_copy(r0.at[pl.ds(0, 112)], acc.at[pl.ds(base + 512, 112)])

    @pl.when(sid == NS - 1)
    def _():
        pltpu.sync_copy(r0.at[pl.ds(0, 16)], acc.at[pl.ds(N - 16, 16)])

    plsc.subcore_barrier()

    def start_cw(c, b):
        pltpu.async_copy(cw_hbm.at[wid].at[c], cwbuf[b], csem[b])

    def wait_cw(b):
        pltpu.make_async_copy(cw_hbm.at[wid].at[0], cwbuf[b], csem[b]).wait()

    def start_row(c, b):
        pltpu.async_copy(row_hbm.at[wid].at[c], ribuf[b], rsem[b])

    def wait_row(b):
        pltpu.make_async_copy(row_hbm.at[wid].at[0], ribuf[b], rsem[b]).wait()

    def start_gather(b):
        pltpu.async_copy(dense_hbm.at[cwbuf[b].at[0]], rbuf[b], gsem[b])

    def wait_gather(b):
        pltpu.make_async_copy(dense_hbm.at[cwbuf[b].at[0]], rbuf[b],
                              gsem[b]).wait()

    def start_scatter(b):
        pltpu.async_copy(rbuf[b], acc.at[ribuf[b].at[0]], ssem[b], add=True)

    def wait_scatter(b):
        pltpu.make_async_copy(rbuf[b], acc.at[ribuf[b].at[0]],
                              ssem[b]).wait()

    def scale(b):
        rv, cw = rbuf[b], cwbuf[b]

        @pl.loop(0, CH, step=L)
        def _(g):
            wg = plsc.bitcast(cw[1, pl.ds(g, L)], jnp.float32)
            for e in range(L):
                idxs = jnp.full((L, 1), e, jnp.int32)
                wv = lax.gather(wg, idxs, _GDN, slice_sizes=(1,),
                                mode=lax.GatherScatterMode.PROMISE_IN_BOUNDS)
                for t in range(D // L):
                    rv[g + e, pl.ds(t * L, L)] = (
                        rv[g + e, pl.ds(t * L, L)] * wv)

    start_cw(0, 0)
    start_row(0, 0)
    wait_cw(0)
    start_gather(0)
    start_cw(1, 1)

    @pl.loop(0, CHUNKS, step=2)
    def _(c):
        for u in range(2):
            cc = c + u
            b = u
            nb = 1 - u

            @pl.when(cc + 1 < CHUNKS)
            def _():
                wait_cw(nb)

            @pl.when(cc >= 1)
            def _():
                wait_scatter(nb)

            @pl.when(cc + 1 < CHUNKS)
            def _():
                start_row(cc + 1, nb)
                start_gather(nb)

            wait_gather(b)
            scale(b)
            wait_row(b)
            start_scatter(b)

            @pl.when(cc + 2 < CHUNKS)
            def _():
                start_cw(cc + 2, b)

    wait_scatter((CHUNKS - 1) % 2)
    plsc.subcore_barrier()

    for k in range(4):
        pltpu.sync_copy(acc.at[pl.ds(base + k * CH, CH)],
                        out_hbm.at[cid].at[pl.ds(base + k * CH, CH)])
    pltpu.sync_copy(acc.at[pl.ds(base + 512, 112)],
                    out_hbm.at[cid].at[pl.ds(base + 512, 112)])

    @pl.when(sid == NS - 1)
    def _():
        pltpu.sync_copy(acc.at[pl.ds(N - 16, 16)],
                        out_hbm.at[cid].at[pl.ds(N - 16, 16)])


def _spmm_partials(dense, cwp, rowp):
    return _spmm_kernel(dense, cwp, rowp)


def kernel(x, edge_index, edge_weight, W1, b1, W2, b2):
    row = edge_index[0].astype(jnp.int32)
    col = edge_index[1].astype(jnp.int32)
    w = edge_weight.astype(jnp.float32)

    pad = E_PAD - E
    zi = jnp.zeros((pad,), jnp.int32)
    colp = jnp.concatenate([col, zi]).reshape(NW, CHUNKS, 1, CH)
    wbits = lax.bitcast_convert_type(
        jnp.concatenate([w, jnp.zeros((pad,), jnp.float32)]), jnp.int32
    ).reshape(NW, CHUNKS, 1, CH)
    cwp = jnp.concatenate([colp, wbits], axis=2)
    rowp = jnp.concatenate([row, zi]).reshape(NW, CHUNKS, 1, CH)

    b1e = jnp.broadcast_to(b1[None, :], (8, D))
    b2e = jnp.broadcast_to(b2[None, :], (8, D))

    support = _matmul(x, W1)
    p1 = _spmm_partials(support, cwp, rowp)
    support2 = _relu_bias_matmul(p1, b1e, W2)
    p2 = _spmm_partials(support2, cwp, rowp)
    return _bias_add(p2, b2e)
